# baseline (speedup 1.0000x reference)
import jax
import jax.numpy as jnp
from jax.experimental import pallas as pl


def _add_pe_flat(x_ref, pe_ref, o_ref, *, blk, per_batch):
    i = pl.program_id(0)
    s = (i % per_batch) * blk
    o_ref[...] = x_ref[...] + pe_ref[pl.ds(s, blk), :]


from functools import partial


def kernel(x, pe, position_ids):
    batch, seq_len, d_model = x.shape
    blk = 512
    per_batch = seq_len // blk
    xf = x.reshape(batch * seq_len, d_model)
    grid = (batch * per_batch,)
    out = pl.pallas_call(
        partial(_add_pe_flat, blk=blk, per_batch=per_batch),
        grid=grid,
        in_specs=[
            pl.BlockSpec((blk, d_model), lambda i: (i, 0)),
            pl.BlockSpec((seq_len, d_model), lambda i: (0, 0)),
        ],
        out_specs=pl.BlockSpec((blk, d_model), lambda i: (i, 0)),
        out_shape=jax.ShapeDtypeStruct(xf.shape, x.dtype),
    )(xf, pe[:seq_len])
    return out.reshape(x.shape)


# speedup vs baseline: 1.1034x; 1.1034x over previous
import jax
import jax.numpy as jnp
from jax.experimental import pallas as pl


def _add_pe_block(x_ref, pe_ref, o_ref):
    o_ref[...] = x_ref[...] + pe_ref[...][None, :, :]


def kernel(x, pe, position_ids):
    batch, seq_len, d_model = x.shape
    blk = 512
    grid = (seq_len // blk,)
    return pl.pallas_call(
        _add_pe_block,
        grid=grid,
        in_specs=[
            pl.BlockSpec((batch, blk, d_model), lambda i: (0, i, 0)),
            pl.BlockSpec((blk, d_model), lambda i: (i, 0)),
        ],
        out_specs=pl.BlockSpec((batch, blk, d_model), lambda i: (0, i, 0)),
        out_shape=jax.ShapeDtypeStruct(x.shape, x.dtype),
    )(x, pe[:seq_len])
